# Initial kernel scaffold; baseline (speedup 1.0000x reference)
import functools

import jax
import jax.numpy as jnp
from jax import lax
from jax.experimental import pallas as pl
from jax.experimental.pallas import tpu as pltpu
from jax.experimental.pallas import tpu_sc as plsc

N = 10000
E = 320000
H = 50
D = 64
NODE_IN = 128
EDGE_IN_DIM = 16

NC, NS = 2, 16
NW = NC * NS
EPW = E // NW
C = 80
NCHUNK = EPW // C
RPT = N // NS
ZR = 125
BLK = 4000

_f32 = jnp.float32


def _pad_lin(p, in_pad, out_pad):
    W, b = p
    W = jnp.pad(W, ((0, in_pad - W.shape[0]), (0, out_pad - W.shape[1])))
    b = jnp.pad(b, (0, out_pad - b.shape[0])).reshape(1, -1)
    return W, b


def _p1_body(nf, wl, bl, wr, br, wn, bn, t_out, nn_out):
    x = nf[...]
    left = x @ wl[...] + bl[...]
    right = x @ wr[...] + br[...]
    t_out[...] = jnp.concatenate([left, right], axis=1)
    nn_out[...] = jnp.maximum(x @ wn[...] + bn[...], 0.0)


def _p1(nf, wl, bl, wr, br, wn, bn):
    return pl.pallas_call(
        _p1_body,
        out_shape=(
            jax.ShapeDtypeStruct((N, 2 * D), _f32),
            jax.ShapeDtypeStruct((N, D), _f32),
        ),
    )(nf, wl, bl, wr, br, wn, bn)


def _p2_body(t_hbm, src_hbm, dst_hbm, fs_hbm, sidx, didx, gs, gd, ob, sem1, sem2):
    c = lax.axis_index("c")
    s = lax.axis_index("s")
    wid = s * NC + c
    base0 = wid * EPW

    def chunk(j, carry):
        base = base0 + j * C
        pltpu.sync_copy(src_hbm.at[pl.ds(base, C)], sidx)
        pltpu.sync_copy(dst_hbm.at[pl.ds(base, C)], didx)
        cp1 = pltpu.async_copy(t_hbm.at[sidx], gs, sem1)
        cp2 = pltpu.async_copy(t_hbm.at[didx], gd, sem2)
        cp1.wait()
        cp2.wait()

        def row(r, carry2):
            for h in range(D // 16):
                a = gs[r, pl.ds(h * 16, 16)]
                b = gd[r, pl.ds(D + h * 16, 16)]
                ob[r, pl.ds(h * 16, 16)] = jnp.maximum(a + b, 0.0)
                a2 = gs[r, pl.ds(D + h * 16, 16)]
                b2 = gd[r, pl.ds(h * 16, 16)]
                ob[r, pl.ds(D + h * 16, 16)] = jnp.maximum(a2 + b2, 0.0)
            return carry2

        lax.fori_loop(0, C, row, 0)
        pltpu.sync_copy(ob, fs_hbm.at[pl.ds(base, C)])
        return carry

    lax.fori_loop(0, NCHUNK, chunk, 0)


def _p2(t, src, dst):
    mesh = plsc.VectorSubcoreMesh(core_axis_name="c", subcore_axis_name="s")
    f = functools.partial(
        pl.kernel,
        out_type=jax.ShapeDtypeStruct((E, 2 * D), _f32),
        mesh=mesh,
        scratch_types=[
            pltpu.VMEM((C,), jnp.int32),
            pltpu.VMEM((C,), jnp.int32),
            pltpu.VMEM((C, 2 * D), _f32),
            pltpu.VMEM((C, 2 * D), _f32),
            pltpu.VMEM((C, 2 * D), _f32),
            pltpu.SemaphoreType.DMA,
            pltpu.SemaphoreType.DMA,
        ],
    )(_p2_body)
    return f(t, src, dst)


def _p3_body(ef, fs, we2e, be2e, u12, u3, bu, we2n0, be2n0, we2n1, be2n1, z0, z1):
    e = ef[...]
    f = fs[...]
    third = jnp.maximum(e @ we2e[...] + be2e[...], 0.0)
    ne = jnp.maximum(f @ u12[...] + third @ u3[...] + bu[...], 0.0)
    z0[...] = jnp.maximum(e @ we2n0[...] + be2n0[...], 0.0)
    z1[...] = jnp.maximum(ne @ we2n1[...] + be2n1[...], 0.0)


def _p3(ef, fs, we2e, be2e, u12, u3, bu, we2n0, be2n0, we2n1, be2n1):
    grid = (E // BLK,)
    full = lambda w: pl.BlockSpec(w.shape, lambda i: (0, 0))
    return pl.pallas_call(
        _p3_body,
        grid=grid,
        in_specs=[
            pl.BlockSpec((BLK, EDGE_IN_DIM), lambda i: (i, 0)),
            pl.BlockSpec((BLK, 2 * D), lambda i: (i, 0)),
            full(we2e), full(be2e), full(u12), full(u3), full(bu),
            full(we2n0), full(be2n0), full(we2n1), full(be2n1),
        ],
        out_specs=(
            pl.BlockSpec((BLK, D), lambda i: (i, 0)),
            pl.BlockSpec((BLK, D), lambda i: (i, 0)),
        ),
        out_shape=(
            jax.ShapeDtypeStruct((E, D), _f32),
            jax.ShapeDtypeStruct((E, D), _f32),
        ),
    )(ef, fs, we2e, be2e, u12, u3, bu, we2n0, be2n0, we2n1, be2n1)


def _p4_body(z0_hbm, z1_hbm, dst_hbm, p0_hbm, p1_hbm,
             didx, b0, b1, zb, acc0, acc1):
    c = lax.axis_index("c")
    s = lax.axis_index("s")
    wid = s * NC + c

    def zrow(r, carry):
        for h in range(D // 16):
            zb[r, pl.ds(h * 16, 16)] = jnp.zeros((16,), _f32)
        return carry

    lax.fori_loop(0, ZR, zrow, 0)

    def zcopy(k, carry):
        off = s * RPT + k * ZR
        pltpu.sync_copy(zb, acc0.at[pl.ds(off, ZR)])
        pltpu.sync_copy(zb, acc1.at[pl.ds(off, ZR)])
        return carry

    lax.fori_loop(0, RPT // ZR, zcopy, 0)
    plsc.subcore_barrier()

    def chunk(j, carry):
        base = wid * EPW + j * C
        pltpu.sync_copy(dst_hbm.at[pl.ds(base, C)], didx)
        pltpu.sync_copy(z0_hbm.at[pl.ds(base, C)], b0)
        pltpu.sync_copy(z1_hbm.at[pl.ds(base, C)], b1)
        pltpu.sync_copy(b0, acc0.at[didx], add=True)
        pltpu.sync_copy(b1, acc1.at[didx], add=True)
        return carry

    lax.fori_loop(0, NCHUNK, chunk, 0)
    plsc.subcore_barrier()

    def wcopy(k, carry):
        off = s * RPT + k * ZR
        pltpu.sync_copy(acc0.at[pl.ds(off, ZR)], p0_hbm.at[c, pl.ds(off, ZR)])
        pltpu.sync_copy(acc1.at[pl.ds(off, ZR)], p1_hbm.at[c, pl.ds(off, ZR)])
        return carry

    lax.fori_loop(0, RPT // ZR, wcopy, 0)


def _p4(z0, z1, dst):
    mesh = plsc.VectorSubcoreMesh(core_axis_name="c", subcore_axis_name="s")
    f = functools.partial(
        pl.kernel,
        out_type=(
            jax.ShapeDtypeStruct((NC, N, D), _f32),
            jax.ShapeDtypeStruct((NC, N, D), _f32),
        ),
        mesh=mesh,
        scratch_types=[
            pltpu.VMEM((C,), jnp.int32),
            pltpu.VMEM((C, D), _f32),
            pltpu.VMEM((C, D), _f32),
            pltpu.VMEM((ZR, D), _f32),
            pltpu.VMEM_SHARED((N, D), _f32),
            pltpu.VMEM_SHARED((N, D), _f32),
        ],
    )(_p4_body)
    return f(z0, z1, dst)


def _p5_body(nn0, p0, p1, a1, a2, b0, wn1, bn1, c1, c2, b1, out):
    en0 = p0[0] + p0[1]
    en1 = p1[0] + p1[1]
    new_node0 = jnp.maximum(nn0[...] @ a1[...] + en0 @ a2[...] + b0[...], 0.0)
    node_node1 = jnp.maximum(new_node0 @ wn1[...] + bn1[...], 0.0)
    out[...] = jnp.maximum(node_node1 @ c1[...] + en1 @ c2[...] + b1[...], 0.0)


def _p5(nn0, p0, p1, a1, a2, b0, wn1, bn1, c1, c2, b1):
    return pl.pallas_call(
        _p5_body,
        out_shape=jax.ShapeDtypeStruct((N, D), _f32),
    )(nn0, p0, p1, a1, a2, b0, wn1, bn1, c1, c2, b1)


def kernel(node_feats, edge_feats, params, edge_index):
    l0, l1 = params[0], params[1]
    src, dst = edge_index[0], edge_index[1]

    wl, bl = _pad_lin(l0['left'], NODE_IN, D)
    wr, br = _pad_lin(l0['right'], NODE_IN, D)
    wn0, bn0 = _pad_lin(l0['n2n'], NODE_IN, D)

    we2e, be2e = _pad_lin(l0['e2e'], EDGE_IN_DIM, D)
    wu_raw, bu_raw = l0['upd_e']
    u1 = jnp.pad(wu_raw[0:H], ((0, D - H), (0, D - H)))
    u2 = jnp.pad(wu_raw[H:2 * H], ((0, D - H), (0, D - H)))
    u3 = jnp.pad(wu_raw[2 * H:3 * H], ((0, D - H), (0, D - H)))
    u12 = jnp.concatenate([u1, u2], axis=0)
    bu = jnp.pad(bu_raw, (0, D - H)).reshape(1, -1)
    we2n0, be2n0 = _pad_lin(l0['e2n'], EDGE_IN_DIM, D)
    we2n1, be2n1 = _pad_lin(l1['e2n'], D, D)

    wun0, bun0 = l0['upd_n']
    a1 = jnp.pad(wun0[0:H], ((0, D - H), (0, D - H)))
    a2 = jnp.pad(wun0[H:2 * H], ((0, D - H), (0, D - H)))
    b0 = jnp.pad(bun0, (0, D - H)).reshape(1, -1)
    wn1, bn1 = _pad_lin(l1['n2n'], D, D)
    wun1, bun1 = l1['upd_n']
    c1 = jnp.pad(wun1[0:H], ((0, D - H), (0, D - H)))
    c2 = jnp.pad(wun1[H:2 * H], ((0, D - H), (0, D - H)))
    b1 = jnp.pad(bun1, (0, D - H)).reshape(1, -1)

    t, nn0 = _p1(node_feats, wl, bl, wr, br, wn0, bn0)
    fs = _p2(t, src, dst)
    z0, z1 = _p3(edge_feats, fs, we2e, be2e, u12, u3, bu,
                 we2n0, be2n0, we2n1, be2n1)
    p0, p1 = _p4(z0, z1, dst)
    out = _p5(nn0, p0, p1, a1, a2, b0, wn1, bn1, c1, c2, b1)
    return out[:, :H]


# speedup vs baseline: 4.3331x; 4.3331x over previous
import functools

import jax
import jax.numpy as jnp
from jax import lax
from jax.experimental import pallas as pl
from jax.experimental.pallas import tpu as pltpu
from jax.experimental.pallas import tpu_sc as plsc

N = 10000
E = 320000
H = 50
D = 64
NODE_IN = 128
EDGE_IN_DIM = 16

NC, NS = 2, 16
NW = NC * NS
EPW = E // NW
C = 80
NCHUNK = EPW // C
RCH = 80
NRCH = N // RCH
CPT = 8
BLK = 4000

_f32 = jnp.float32


def _pad_lin(p, in_pad, out_pad):
    W, b = p
    W = jnp.pad(W, ((0, in_pad - W.shape[0]), (0, out_pad - W.shape[1])))
    b = jnp.pad(b, (0, out_pad - b.shape[0])).reshape(1, -1)
    return W, b


def _p1_body(nf, wl, bl, wr, br, wn, bn, t_out, nn_out):
    x = nf[...]
    left = x @ wl[...] + bl[...]
    right = x @ wr[...] + br[...]
    t_out[...] = jnp.concatenate([left, right], axis=1)
    nn_out[...] = jnp.maximum(x @ wn[...] + bn[...], 0.0)


def _p1(nf, wl, bl, wr, br, wn, bn):
    return pl.pallas_call(
        _p1_body,
        out_shape=(
            jax.ShapeDtypeStruct((N, 2 * D), _f32),
            jax.ShapeDtypeStruct((N, D), _f32),
        ),
    )(nf, wl, bl, wr, br, wn, bn)


def _p2_body(t_hbm, src_hbm, dst_hbm, fs_hbm, sidx, didx, gs, gd, ob, sem1, sem2):
    c = lax.axis_index("c")
    s = lax.axis_index("s")
    wid = s * NC + c
    base0 = wid * EPW

    def chunk(j, carry):
        base = base0 + j * C
        pltpu.sync_copy(src_hbm.at[pl.ds(base, C)], sidx)
        pltpu.sync_copy(dst_hbm.at[pl.ds(base, C)], didx)
        cp1 = pltpu.async_copy(t_hbm.at[sidx], gs, sem1)
        cp2 = pltpu.async_copy(t_hbm.at[didx], gd, sem2)
        cp1.wait()
        cp2.wait()

        def row(r, carry2):
            for h in range(D // 16):
                a = gs[r, pl.ds(h * 16, 16)]
                b = gd[r, pl.ds(D + h * 16, 16)]
                ob[r, pl.ds(h * 16, 16)] = jnp.maximum(a + b, 0.0)
                a2 = gs[r, pl.ds(D + h * 16, 16)]
                b2 = gd[r, pl.ds(h * 16, 16)]
                ob[r, pl.ds(D + h * 16, 16)] = jnp.maximum(a2 + b2, 0.0)
            return carry2

        lax.fori_loop(0, C, row, 0)
        pltpu.sync_copy(ob, fs_hbm.at[pl.ds(base, C)])
        return carry

    lax.fori_loop(0, NCHUNK, chunk, 0)


def _p2(t, src, dst):
    mesh = plsc.VectorSubcoreMesh(core_axis_name="c", subcore_axis_name="s")
    f = functools.partial(
        pl.kernel,
        out_type=jax.ShapeDtypeStruct((E, 2 * D), _f32),
        mesh=mesh,
        scratch_types=[
            pltpu.VMEM((C,), jnp.int32),
            pltpu.VMEM((C,), jnp.int32),
            pltpu.VMEM((C, 2 * D), _f32),
            pltpu.VMEM((C, 2 * D), _f32),
            pltpu.VMEM((C, 2 * D), _f32),
            pltpu.SemaphoreType.DMA,
            pltpu.SemaphoreType.DMA,
        ],
    )(_p2_body)
    return f(t, src, dst)


def _p3_body(ef, fs, we2e, be2e, u12, u3, bu, we2n0, be2n0, we2n1, be2n1, z0):
    e = ef[...]
    f = fs[...]
    third = jnp.maximum(e @ we2e[...] + be2e[...], 0.0)
    ne = jnp.maximum(f @ u12[...] + third @ u3[...] + bu[...], 0.0)
    za = jnp.maximum(e @ we2n0[...] + be2n0[...], 0.0)
    zb = jnp.maximum(ne @ we2n1[...] + be2n1[...], 0.0)
    z0[...] = jnp.concatenate([za, zb], axis=1)


def _p3(ef, fs, we2e, be2e, u12, u3, bu, we2n0, be2n0, we2n1, be2n1):
    grid = (E // BLK,)
    full = lambda w: pl.BlockSpec(w.shape, lambda i: (0, 0))
    return pl.pallas_call(
        _p3_body,
        grid=grid,
        in_specs=[
            pl.BlockSpec((BLK, EDGE_IN_DIM), lambda i: (i, 0)),
            pl.BlockSpec((BLK, 2 * D), lambda i: (i, 0)),
            full(we2e), full(be2e), full(u12), full(u3), full(bu),
            full(we2n0), full(be2n0), full(we2n1), full(be2n1),
        ],
        out_specs=pl.BlockSpec((BLK, 2 * D), lambda i: (i, 0)),
        out_shape=jax.ShapeDtypeStruct((E, 2 * D), _f32),
    )(ef, fs, we2e, be2e, u12, u3, bu, we2n0, be2n0, we2n1, be2n1)


def _p4_body(z_hbm, dst_hbm, p_hbm, didx, b, zb, acc):
    c = lax.axis_index("c")
    s = lax.axis_index("s")
    wid = s * NC + c

    start = s * CPT
    ncop = jnp.minimum(CPT, NRCH - start)

    def zrow(r, carry):
        for h in range(2 * D // 16):
            zb[r, pl.ds(h * 16, 16)] = jnp.zeros((16,), _f32)
        return carry

    lax.fori_loop(0, RCH, zrow, 0)

    def zcopy(k, carry):
        off = (start + k) * RCH
        pltpu.sync_copy(zb, acc.at[pl.ds(off, RCH)])
        return carry

    lax.fori_loop(0, ncop, zcopy, 0)
    plsc.subcore_barrier()

    def chunk(j, carry):
        base = wid * EPW + j * C
        pltpu.sync_copy(dst_hbm.at[pl.ds(base, C)], didx)
        pltpu.sync_copy(z_hbm.at[pl.ds(base, C)], b)
        pltpu.sync_copy(b, acc.at[didx], add=True)
        return carry

    lax.fori_loop(0, NCHUNK, chunk, 0)
    plsc.subcore_barrier()

    def wcopy(k, carry):
        off = (start + k) * RCH
        pltpu.sync_copy(acc.at[pl.ds(off, RCH)], zb)
        pltpu.sync_copy(zb, p_hbm.at[c, pl.ds(off, RCH)])
        return carry

    lax.fori_loop(0, ncop, wcopy, 0)


def _p4(z, dst):
    mesh = plsc.VectorSubcoreMesh(core_axis_name="c", subcore_axis_name="s")
    f = functools.partial(
        pl.kernel,
        out_type=jax.ShapeDtypeStruct((NC, N, 2 * D), _f32),
        mesh=mesh,
        scratch_types=[
            pltpu.VMEM((C,), jnp.int32),
            pltpu.VMEM((C, 2 * D), _f32),
            pltpu.VMEM((RCH, 2 * D), _f32),
            pltpu.VMEM_SHARED((N, 2 * D), _f32),
        ],
    )(_p4_body)
    return f(z, dst)


def _p5_body(nn0, p, a1, a2, b0, wn1, bn1, c1, c2, b1, out):
    psum = p[0] + p[1]
    en0 = psum[:, :D]
    en1 = psum[:, D:]
    new_node0 = jnp.maximum(nn0[...] @ a1[...] + en0 @ a2[...] + b0[...], 0.0)
    node_node1 = jnp.maximum(new_node0 @ wn1[...] + bn1[...], 0.0)
    out[...] = jnp.maximum(node_node1 @ c1[...] + en1 @ c2[...] + b1[...], 0.0)


def _p5(nn0, p, a1, a2, b0, wn1, bn1, c1, c2, b1):
    return pl.pallas_call(
        _p5_body,
        out_shape=jax.ShapeDtypeStruct((N, D), _f32),
    )(nn0, p, a1, a2, b0, wn1, bn1, c1, c2, b1)


def kernel(node_feats, edge_feats, params, edge_index):
    l0, l1 = params[0], params[1]
    src, dst = edge_index[0], edge_index[1]

    wl, bl = _pad_lin(l0['left'], NODE_IN, D)
    wr, br = _pad_lin(l0['right'], NODE_IN, D)
    wn0, bn0 = _pad_lin(l0['n2n'], NODE_IN, D)

    we2e, be2e = _pad_lin(l0['e2e'], EDGE_IN_DIM, D)
    wu_raw, bu_raw = l0['upd_e']
    u1 = jnp.pad(wu_raw[0:H], ((0, D - H), (0, D - H)))
    u2 = jnp.pad(wu_raw[H:2 * H], ((0, D - H), (0, D - H)))
    u3 = jnp.pad(wu_raw[2 * H:3 * H], ((0, D - H), (0, D - H)))
    u12 = jnp.concatenate([u1, u2], axis=0)
    bu = jnp.pad(bu_raw, (0, D - H)).reshape(1, -1)
    we2n0, be2n0 = _pad_lin(l0['e2n'], EDGE_IN_DIM, D)
    we2n1, be2n1 = _pad_lin(l1['e2n'], D, D)

    wun0, bun0 = l0['upd_n']
    a1 = jnp.pad(wun0[0:H], ((0, D - H), (0, D - H)))
    a2 = jnp.pad(wun0[H:2 * H], ((0, D - H), (0, D - H)))
    b0 = jnp.pad(bun0, (0, D - H)).reshape(1, -1)
    wn1, bn1 = _pad_lin(l1['n2n'], D, D)
    wun1, bun1 = l1['upd_n']
    c1 = jnp.pad(wun1[0:H], ((0, D - H), (0, D - H)))
    c2 = jnp.pad(wun1[H:2 * H], ((0, D - H), (0, D - H)))
    b1 = jnp.pad(bun1, (0, D - H)).reshape(1, -1)

    t, nn0 = _p1(node_feats, wl, bl, wr, br, wn0, bn0)
    fs = _p2(t, src, dst)
    z = _p3(edge_feats, fs, we2e, be2e, u12, u3, bu,
            we2n0, be2n0, we2n1, be2n1)
    p = _p4(z, dst)
    out = _p5(nn0, p, a1, a2, b0, wn1, bn1, c1, c2, b1)
    return out[:, :H]


# speedup vs baseline: 6.7406x; 1.5556x over previous
import functools

import jax
import jax.numpy as jnp
from jax import lax
from jax.experimental import pallas as pl
from jax.experimental.pallas import tpu as pltpu
from jax.experimental.pallas import tpu_sc as plsc

N = 10000
E = 320000
H = 50
D = 64
NODE_IN = 128
EDGE_IN_DIM = 16

NC, NS = 2, 16
NW = NC * NS
C = 80
RCH = 80
NRCH = N // RCH
CPT = 8
BLK = 2560

EA = 163840
EB = E - EA

_f32 = jnp.float32


def _pad_lin(p, in_pad, out_pad):
    W, b = p
    W = jnp.pad(W, ((0, in_pad - W.shape[0]), (0, out_pad - W.shape[1])))
    b = jnp.pad(b, (0, out_pad - b.shape[0])).reshape(1, -1)
    return W, b


def _p1_body(nf, wl, bl, wr, br, wn, bn, t_out, nn_out):
    x = nf[...]
    left = x @ wl[...] + bl[...]
    right = x @ wr[...] + br[...]
    t_out[...] = jnp.concatenate([left, right], axis=1)
    nn_out[...] = jnp.maximum(x @ wn[...] + bn[...], 0.0)


def _p1(nf, wl, bl, wr, br, wn, bn):
    return pl.pallas_call(
        _p1_body,
        out_shape=(
            jax.ShapeDtypeStruct((N, 2 * D), _f32),
            jax.ShapeDtypeStruct((N, D), _f32),
        ),
    )(nf, wl, bl, wr, br, wn, bn)


@functools.lru_cache(maxsize=None)
def _make_p2(n_edges):
    epw = n_edges // NW
    nchunk = epw // C
    odd = nchunk % 2 == 1

    def body(t_hbm, src3_hbm, dst3_hbm, fs_hbm,
             sidx_all, didx_all,
             gs0, gd0, ob0, ss0, sd0, os0,
             gs1, gd1, ob1, ss1, sd1, os1):
        c = lax.axis_index("c")
        s = lax.axis_index("s")
        wid = s * NC + c
        base0 = wid * epw
        sets = ((gs0, gd0, ob0, ss0, sd0, os0),
                (gs1, gd1, ob1, ss1, sd1, os1))

        pltpu.sync_copy(src3_hbm.at[wid], sidx_all)
        pltpu.sync_copy(dst3_hbm.at[wid], didx_all)

        def start(k, S):
            pltpu.async_copy(t_hbm.at[sidx_all.at[k]], S[0], S[3])
            pltpu.async_copy(t_hbm.at[didx_all.at[k]], S[1], S[4])

        def wait(S):
            pltpu.make_async_copy(t_hbm.at[sidx_all.at[0]], S[0], S[3]).wait()
            pltpu.make_async_copy(t_hbm.at[didx_all.at[0]], S[1], S[4]).wait()

        def compute_write(k, S):
            gs, gd, ob, osem = S[0], S[1], S[2], S[5]

            @pl.when(k >= 2)
            def _():
                pltpu.make_async_copy(ob, fs_hbm.at[pl.ds(0, C)], osem).wait()

            def row(r, carry2):
                for h in range(D // 16):
                    a = gs[r, pl.ds(h * 16, 16)]
                    b = gd[r, pl.ds(D + h * 16, 16)]
                    ob[r, pl.ds(h * 16, 16)] = jnp.maximum(a + b, 0.0)
                    a2 = gs[r, pl.ds(D + h * 16, 16)]
                    b2 = gd[r, pl.ds(h * 16, 16)]
                    ob[r, pl.ds(D + h * 16, 16)] = jnp.maximum(a2 + b2, 0.0)
                return carry2

            lax.fori_loop(0, C, row, 0)
            pltpu.async_copy(ob, fs_hbm.at[pl.ds(base0 + k * C, C)], osem)

        start(0, sets[0])

        def lbody(tt, carry):
            j0 = 2 * tt
            wait(sets[0])
            start(j0 + 1, sets[1])
            compute_write(j0, sets[0])
            wait(sets[1])

            @pl.when(j0 + 2 < nchunk)
            def _():
                start(j0 + 2, sets[0])

            compute_write(j0 + 1, sets[1])
            return carry

        lax.fori_loop(0, nchunk // 2, lbody, 0)
        if odd:
            wait(sets[0])
            compute_write(nchunk - 1, sets[0])
        pltpu.make_async_copy(ob0, fs_hbm.at[pl.ds(0, C)], os0).wait()
        pltpu.make_async_copy(ob1, fs_hbm.at[pl.ds(0, C)], os1).wait()

    mesh = plsc.VectorSubcoreMesh(core_axis_name="c", subcore_axis_name="s")
    one_set = [
        pltpu.VMEM((C, 2 * D), _f32),
        pltpu.VMEM((C, 2 * D), _f32),
        pltpu.VMEM((C, 2 * D), _f32),
        pltpu.SemaphoreType.DMA,
        pltpu.SemaphoreType.DMA,
        pltpu.SemaphoreType.DMA,
    ]
    return functools.partial(
        pl.kernel,
        out_type=jax.ShapeDtypeStruct((n_edges, 2 * D), _f32),
        mesh=mesh,
        scratch_types=[pltpu.VMEM((nchunk, C), jnp.int32),
                       pltpu.VMEM((nchunk, C), jnp.int32)]
        + one_set + one_set,
    )(body)


def _p3_body(ef, fs, we2e, be2e, u12, u3, bu, we2n0, be2n0, we2n1, be2n1, z0):
    e = ef[...]
    f = fs[...]
    third = jnp.maximum(e @ we2e[...] + be2e[...], 0.0)
    ne = jnp.maximum(f @ u12[...] + third @ u3[...] + bu[...], 0.0)
    za = jnp.maximum(e @ we2n0[...] + be2n0[...], 0.0)
    zb = jnp.maximum(ne @ we2n1[...] + be2n1[...], 0.0)
    z0[...] = jnp.concatenate([za, zb], axis=1)


def _p3(ef, fs, we2e, be2e, u12, u3, bu, we2n0, be2n0, we2n1, be2n1):
    n_edges = ef.shape[0]
    grid = (n_edges // BLK,)
    full = lambda w: pl.BlockSpec(w.shape, lambda i: (0, 0))
    return pl.pallas_call(
        _p3_body,
        grid=grid,
        in_specs=[
            pl.BlockSpec((BLK, EDGE_IN_DIM), lambda i: (i, 0)),
            pl.BlockSpec((BLK, 2 * D), lambda i: (i, 0)),
            full(we2e), full(be2e), full(u12), full(u3), full(bu),
            full(we2n0), full(be2n0), full(we2n1), full(be2n1),
        ],
        out_specs=pl.BlockSpec((BLK, 2 * D), lambda i: (i, 0)),
        out_shape=jax.ShapeDtypeStruct((n_edges, 2 * D), _f32),
    )(ef, fs, we2e, be2e, u12, u3, bu, we2n0, be2n0, we2n1, be2n1)


@functools.lru_cache(maxsize=None)
def _make_p4(n_edges):
    epw = n_edges // NW
    nchunk = epw // C
    odd = nchunk % 2 == 1

    def body(z_hbm, dst3_hbm, p_hbm, didx_all, b0, sm0, b1, sm1, zb, acc):
        c = lax.axis_index("c")
        s = lax.axis_index("s")
        wid = s * NC + c

        rstart = s * CPT
        ncop = jnp.minimum(CPT, NRCH - rstart)

        def zrow(r, carry):
            for h in range(2 * D // 16):
                zb[r, pl.ds(h * 16, 16)] = jnp.zeros((16,), _f32)
            return carry

        lax.fori_loop(0, RCH, zrow, 0)

        def zcopy(k, carry):
            off = (rstart + k) * RCH
            pltpu.sync_copy(zb, acc.at[pl.ds(off, RCH)])
            return carry

        lax.fori_loop(0, ncop, zcopy, 0)
        plsc.subcore_barrier()

        pltpu.sync_copy(dst3_hbm.at[wid], didx_all)
        sets = ((b0, sm0), (b1, sm1))

        def start(k, S):
            base = wid * epw + k * C
            pltpu.async_copy(z_hbm.at[pl.ds(base, C)], S[0], S[1])

        def wait(S):
            pltpu.make_async_copy(z_hbm.at[pl.ds(0, C)], S[0], S[1]).wait()

        def scat(k, S):
            pltpu.sync_copy(S[0], acc.at[didx_all.at[k]], add=True)

        start(0, sets[0])

        def lbody(tt, carry):
            j0 = 2 * tt
            wait(sets[0])
            start(j0 + 1, sets[1])
            scat(j0, sets[0])
            wait(sets[1])

            @pl.when(j0 + 2 < nchunk)
            def _():
                start(j0 + 2, sets[0])

            scat(j0 + 1, sets[1])
            return carry

        lax.fori_loop(0, nchunk // 2, lbody, 0)
        if odd:
            wait(sets[0])
            scat(nchunk - 1, sets[0])
        plsc.subcore_barrier()

        def wcopy(k, carry):
            off = (rstart + k) * RCH
            pltpu.sync_copy(acc.at[pl.ds(off, RCH)], zb)
            pltpu.sync_copy(zb, p_hbm.at[c, pl.ds(off, RCH)])
            return carry

        lax.fori_loop(0, ncop, wcopy, 0)

    mesh = plsc.VectorSubcoreMesh(core_axis_name="c", subcore_axis_name="s")
    return functools.partial(
        pl.kernel,
        out_type=jax.ShapeDtypeStruct((NC, N, 2 * D), _f32),
        mesh=mesh,
        scratch_types=[
            pltpu.VMEM((nchunk, C), jnp.int32),
            pltpu.VMEM((C, 2 * D), _f32),
            pltpu.SemaphoreType.DMA,
            pltpu.VMEM((C, 2 * D), _f32),
            pltpu.SemaphoreType.DMA,
            pltpu.VMEM((RCH, 2 * D), _f32),
            pltpu.VMEM_SHARED((N, 2 * D), _f32),
        ],
    )(body)


def _p5_body(nn0, pa, pb, a1, a2, b0, wn1, bn1, c1, c2, b1, out):
    psum = pa[0] + pa[1] + pb[0] + pb[1]
    en0 = psum[:, :D]
    en1 = psum[:, D:]
    new_node0 = jnp.maximum(nn0[...] @ a1[...] + en0 @ a2[...] + b0[...], 0.0)
    node_node1 = jnp.maximum(new_node0 @ wn1[...] + bn1[...], 0.0)
    out[...] = jnp.maximum(node_node1 @ c1[...] + en1 @ c2[...] + b1[...], 0.0)


def _p5(nn0, pa, pb, a1, a2, b0, wn1, bn1, c1, c2, b1):
    return pl.pallas_call(
        _p5_body,
        out_shape=jax.ShapeDtypeStruct((N, D), _f32),
    )(nn0, pa, pb, a1, a2, b0, wn1, bn1, c1, c2, b1)


def kernel(node_feats, edge_feats, params, edge_index):
    l0, l1 = params[0], params[1]
    src, dst = edge_index[0], edge_index[1]

    wl, bl = _pad_lin(l0['left'], NODE_IN, D)
    wr, br = _pad_lin(l0['right'], NODE_IN, D)
    wn0, bn0 = _pad_lin(l0['n2n'], NODE_IN, D)

    we2e, be2e = _pad_lin(l0['e2e'], EDGE_IN_DIM, D)
    wu_raw, bu_raw = l0['upd_e']
    u1 = jnp.pad(wu_raw[0:H], ((0, D - H), (0, D - H)))
    u2 = jnp.pad(wu_raw[H:2 * H], ((0, D - H), (0, D - H)))
    u3 = jnp.pad(wu_raw[2 * H:3 * H], ((0, D - H), (0, D - H)))
    u12 = jnp.concatenate([u1, u2], axis=0)
    bu = jnp.pad(bu_raw, (0, D - H)).reshape(1, -1)
    we2n0, be2n0 = _pad_lin(l0['e2n'], EDGE_IN_DIM, D)
    we2n1, be2n1 = _pad_lin(l1['e2n'], D, D)

    wun0, bun0 = l0['upd_n']
    a1 = jnp.pad(wun0[0:H], ((0, D - H), (0, D - H)))
    a2 = jnp.pad(wun0[H:2 * H], ((0, D - H), (0, D - H)))
    b0 = jnp.pad(bun0, (0, D - H)).reshape(1, -1)
    wn1, bn1 = _pad_lin(l1['n2n'], D, D)
    wun1, bun1 = l1['upd_n']
    c1 = jnp.pad(wun1[0:H], ((0, D - H), (0, D - H)))
    c2 = jnp.pad(wun1[H:2 * H], ((0, D - H), (0, D - H)))
    b1 = jnp.pad(bun1, (0, D - H)).reshape(1, -1)

    t, nn0 = _p1(node_feats, wl, bl, wr, br, wn0, bn0)

    edge_w = (we2e, be2e, u12, u3, bu, we2n0, be2n0, we2n1, be2n1)
    parts = []
    for lo, ne in ((0, EA), (EA, EB)):
        s3 = src[lo:lo + ne].reshape(NW, ne // NW // C, C)
        d3 = dst[lo:lo + ne].reshape(NW, ne // NW // C, C)
        fs = _make_p2(ne)(t, s3, d3)
        z = _p3(edge_feats[lo:lo + ne], fs, *edge_w)
        parts.append(_make_p4(ne)(z, d3))

    out = _p5(nn0, parts[0], parts[1], a1, a2, b0, wn1, bn1, c1, c2, b1)
    return out[:, :H]
